# baseline (speedup 1.0000x reference)
import jax
import jax.numpy as jnp
from jax import lax
from jax.experimental import pallas as pl
from jax.experimental.pallas import tpu as pltpu
from jax.experimental.pallas import tpu_sc as plsc

B, T_HALF, D = 4, 1024, 1024
T = 2 * T_HALF
NW = 32
ROWS_PER_W = T // NW
SUB = 32
NSUB = ROWS_PER_W // SUB
LANES = 16


def _sc_body(seq_a, seq_b, emb, out, tab_v, buf0, buf1,
             sem_t, sem_l0, sem_l1, sem_o0, sem_o1):
    cid = lax.axis_index("c")
    sid = lax.axis_index("s")
    wid = sid * 2 + cid
    half = wid // 16
    r0 = (wid % 16) * ROWS_PER_W
    g0 = wid * ROWS_PER_W
    bufs = (buf0, buf1)
    sem_l = (sem_l0, sem_l1)
    sem_o = (sem_o0, sem_o1)

    def load_seq(b, s, k):
        @pl.when(half == 0)
        def _():
            pltpu.async_copy(seq_a.at[b, pl.ds(r0 + SUB * s, SUB), :],
                             bufs[k], sem_l[k])

        @pl.when(half == 1)
        def _():
            pltpu.async_copy(seq_b.at[b, pl.ds(r0 + SUB * s, SUB), :],
                             bufs[k], sem_l[k])

    def wait_seq(b, s, k):
        pltpu.make_async_copy(seq_a.at[b, pl.ds(r0 + SUB * s, SUB), :],
                              bufs[k], sem_l[k]).wait()

    def store_out(b, s, k):
        pltpu.async_copy(bufs[k], out.at[b, pl.ds(g0 + SUB * s, SUB), :],
                         sem_o[k])

    def wait_out(b, s, k):
        pltpu.make_async_copy(bufs[k], out.at[b, pl.ds(g0 + SUB * s, SUB), :],
                              sem_o[k]).wait()

    def add_table(k):
        buf = bufs[k]

        def row(r, carry):
            for j in range(D // LANES):
                plsc.addupdate(buf.at[r, pl.ds(j * LANES, LANES)],
                               tab_v[r, pl.ds(j * LANES, LANES)])
            return carry

        lax.fori_loop(0, SUB, row, 0)

    last_store = [None, None]

    for s in range(NSUB):
        pltpu.async_copy(emb.at[pl.ds(g0 + SUB * s, SUB), :], tab_v, sem_t)
        if last_store[0] is not None:
            wait_out(*last_store[0])
            last_store[0] = None
        load_seq(0, s, 0)
        pltpu.make_async_copy(emb.at[pl.ds(g0 + SUB * s, SUB), :],
                              tab_v, sem_t).wait()
        for b in range(B):
            k = b % 2
            if b + 1 < B:
                kn = (b + 1) % 2
                if last_store[kn] is not None:
                    wait_out(*last_store[kn])
                    last_store[kn] = None
                load_seq(b + 1, s, kn)
            wait_seq(b, s, k)
            add_table(k)
            store_out(b, s, k)
            last_store[k] = (b, s, k)

    for k in range(2):
        if last_store[k] is not None:
            wait_out(*last_store[k])


def kernel(seq_a, seq_b, emb_table):
    mesh = plsc.VectorSubcoreMesh(core_axis_name="c", subcore_axis_name="s")
    f = pl.kernel(
        _sc_body,
        out_type=jax.ShapeDtypeStruct((B, T, D), jnp.float32),
        mesh=mesh,
        scratch_types=[
            pltpu.VMEM((SUB, D), jnp.float32),
            pltpu.VMEM((SUB, D), jnp.float32),
            pltpu.VMEM((SUB, D), jnp.float32),
            pltpu.SemaphoreType.DMA,
            pltpu.SemaphoreType.DMA,
            pltpu.SemaphoreType.DMA,
            pltpu.SemaphoreType.DMA,
            pltpu.SemaphoreType.DMA,
        ],
    )
    return f(seq_a, seq_b, emb_table)


# speedup vs baseline: 1.5362x; 1.5362x over previous
import jax
import jax.numpy as jnp
from jax import lax
from jax.experimental import pallas as pl
from jax.experimental.pallas import tpu as pltpu
from jax.experimental.pallas import tpu_sc as plsc

B, T_HALF, D = 4, 1024, 1024
T = 2 * T_HALF
NW = 32
ROWS_PER_W = T // NW
SUB = 32
NSUB = ROWS_PER_W // SUB
LANES = 16


def _sc_body(seq_a, seq_b, emb, out, tab_v, buf0, buf1,
             sem_t, sem_l0, sem_l1, sem_o0, sem_o1):
    cid = lax.axis_index("c")
    sid = lax.axis_index("s")
    wid = sid * 2 + cid
    half = wid // 16
    r0 = (wid % 16) * ROWS_PER_W
    g0 = wid * ROWS_PER_W
    bufs = (buf0, buf1)
    sem_l = (sem_l0, sem_l1)
    sem_o = (sem_o0, sem_o1)

    def load_seq(b, s, k):
        @pl.when(half == 0)
        def _():
            pltpu.async_copy(seq_a.at[b, pl.ds(r0 + SUB * s, SUB), :],
                             bufs[k], sem_l[k])

        @pl.when(half == 1)
        def _():
            pltpu.async_copy(seq_b.at[b, pl.ds(r0 + SUB * s, SUB), :],
                             bufs[k], sem_l[k])

    def wait_seq(b, s, k):
        pltpu.make_async_copy(seq_a.at[b, pl.ds(r0 + SUB * s, SUB), :],
                              bufs[k], sem_l[k]).wait()

    def store_out(b, s, k):
        pltpu.async_copy(bufs[k], out.at[b, pl.ds(g0 + SUB * s, SUB), :],
                         sem_o[k])

    def wait_out(b, s, k):
        pltpu.make_async_copy(bufs[k], out.at[b, pl.ds(g0 + SUB * s, SUB), :],
                              sem_o[k]).wait()

    def add_table(k):
        buf = bufs[k]

        def row(r, carry):
            for j in range(D // LANES):
                sl = pl.ds(j * LANES, LANES)
                buf[r, sl] = buf[r, sl] + tab_v[r, sl]
            return carry

        lax.fori_loop(0, SUB, row, 0)

    last_store = [None, None]

    for s in range(NSUB):
        pltpu.async_copy(emb.at[pl.ds(g0 + SUB * s, SUB), :], tab_v, sem_t)
        if last_store[0] is not None:
            wait_out(*last_store[0])
            last_store[0] = None
        load_seq(0, s, 0)
        pltpu.make_async_copy(emb.at[pl.ds(g0 + SUB * s, SUB), :],
                              tab_v, sem_t).wait()
        for b in range(B):
            k = b % 2
            if b + 1 < B:
                kn = (b + 1) % 2
                if last_store[kn] is not None:
                    wait_out(*last_store[kn])
                    last_store[kn] = None
                load_seq(b + 1, s, kn)
            wait_seq(b, s, k)
            add_table(k)
            store_out(b, s, k)
            last_store[k] = (b, s, k)

    for k in range(2):
        if last_store[k] is not None:
            wait_out(*last_store[k])


def kernel(seq_a, seq_b, emb_table):
    mesh = plsc.VectorSubcoreMesh(core_axis_name="c", subcore_axis_name="s")
    f = pl.kernel(
        _sc_body,
        out_type=jax.ShapeDtypeStruct((B, T, D), jnp.float32),
        mesh=mesh,
        scratch_types=[
            pltpu.VMEM((SUB, D), jnp.float32),
            pltpu.VMEM((SUB, D), jnp.float32),
            pltpu.VMEM((SUB, D), jnp.float32),
            pltpu.SemaphoreType.DMA,
            pltpu.SemaphoreType.DMA,
            pltpu.SemaphoreType.DMA,
            pltpu.SemaphoreType.DMA,
            pltpu.SemaphoreType.DMA,
        ],
    )
    return f(seq_a, seq_b, emb_table)
